# Initial kernel scaffold; baseline (speedup 1.0000x reference)
import jax
import jax.numpy as jnp
from jax.experimental import pallas as pl


def kernel(x, relative_pos):
    raise NotImplementedError("write your pallas kernel here")


# speedup vs baseline: 41.1364x; 41.1364x over previous
import jax
import jax.numpy as jnp
from jax.experimental import pallas as pl
from jax.experimental.pallas import tpu as pltpu

_H = 32
_NH = 12
_S = _H * _H
_M = 2 * _H - 1


def _bias_body(rp_ref, bias_ref):
    rp = rp_ref[0]
    rp64 = jnp.pad(rp, ((0, 1), (0, 1)))

    r = jax.lax.broadcasted_iota(jnp.int32, (_S, 64), 0)
    a = jax.lax.broadcasted_iota(jnp.int32, (_S, 64), 1)
    oi = (a == (r // _H - r % _H + (_H - 1))).astype(jnp.float32)
    c = jax.lax.broadcasted_iota(jnp.int32, (64, _S), 1)
    b = jax.lax.broadcasted_iota(jnp.int32, (64, _S), 0)
    ojt = (b == (c // _H - c % _H + (_H - 1))).astype(jnp.float32)

    t1 = jnp.dot(oi, rp64, preferred_element_type=jnp.float32)
    t2 = jnp.dot(t1, ojt, preferred_element_type=jnp.float32)
    t4 = t2.reshape(_H, _H, _H, _H).transpose(0, 2, 1, 3)
    bias_ref[0] = t4.reshape(_S, _S)


def _add_body(x_ref, bias_ref, o_ref):
    o_ref[...] = x_ref[...] + bias_ref[...]


def kernel(x, relative_pos):
    bias = pl.pallas_call(
        _bias_body,
        grid=(_NH,),
        in_specs=[pl.BlockSpec((1, _M, _M), lambda h: (h, 0, 0))],
        out_specs=pl.BlockSpec((1, _S, _S), lambda h: (h, 0, 0)),
        out_shape=jax.ShapeDtypeStruct((_NH, _S, _S), jnp.float32),
    )(relative_pos)

    out = pl.pallas_call(
        _add_body,
        grid=(_NH, x.shape[0]),
        in_specs=[
            pl.BlockSpec((1, 1, _S, _S), lambda h, b: (b, h, 0, 0)),
            pl.BlockSpec((1, _S, _S), lambda h, b: (h, 0, 0)),
        ],
        out_specs=pl.BlockSpec((1, 1, _S, _S), lambda h, b: (b, h, 0, 0)),
        out_shape=jax.ShapeDtypeStruct(x.shape, x.dtype),
    )(x, bias)
    return out


# speedup vs baseline: 45.5771x; 1.1079x over previous
import jax
import jax.numpy as jnp
from jax.experimental import pallas as pl
from jax.experimental.pallas import tpu as pltpu

_H = 32
_NH = 12
_S = _H * _H
_M = 2 * _H - 1


def _fused_body(rp_ref, x_ref, o_ref, bias_ref):
    @pl.when(pl.program_id(1) == 0)
    def _build_bias():
        rp64 = jnp.pad(rp_ref[0], ((0, 1), (0, 1)))
        r = jax.lax.broadcasted_iota(jnp.int32, (_S, 64), 0)
        a = jax.lax.broadcasted_iota(jnp.int32, (_S, 64), 1)
        oi = (a == (r // _H - r % _H + (_H - 1))).astype(jnp.float32)
        c = jax.lax.broadcasted_iota(jnp.int32, (64, _S), 1)
        b = jax.lax.broadcasted_iota(jnp.int32, (64, _S), 0)
        ojt = (b == (c // _H - c % _H + (_H - 1))).astype(jnp.float32)
        t1 = jnp.dot(oi, rp64, preferred_element_type=jnp.float32)
        t2 = jnp.dot(t1, ojt, preferred_element_type=jnp.float32)
        t4 = t2.reshape(_H, _H, _H, _H).transpose(0, 2, 1, 3)
        bias_ref[...] = t4.reshape(_S, _S)

    o_ref[0, 0] = x_ref[0, 0] + bias_ref[...]


def kernel(x, relative_pos):
    return pl.pallas_call(
        _fused_body,
        grid=(_NH, x.shape[0]),
        in_specs=[
            pl.BlockSpec((1, _M, _M), lambda h, b: (h, 0, 0)),
            pl.BlockSpec((1, 1, _S, _S), lambda h, b: (b, h, 0, 0)),
        ],
        out_specs=pl.BlockSpec((1, 1, _S, _S), lambda h, b: (b, h, 0, 0)),
        out_shape=jax.ShapeDtypeStruct(x.shape, x.dtype),
        scratch_shapes=[pltpu.VMEM((_S, _S), jnp.float32)],
    )(relative_pos, x)
